# baseline (speedup 1.0000x reference)
import numpy as np
import jax
import jax.numpy as jnp
from jax import lax
from jax.experimental import pallas as pl
from jax.experimental.pallas import tpu as pltpu
from jax.experimental.pallas import tpu_sc as plsc

VOCAB = 100000
D_MODEL = 768
MAX_LEN = 8192
BATCH = 4
SEQ_LEN = 4096
PAD_IDX = 1

TOK = BATCH * SEQ_LEN
NUM_WORKERS = 32
TPW = TOK // NUM_WORKERS
NPASS = 4
POS_PER_W = SEQ_LEN // NPASS // NUM_WORKERS
PCHUNK = 8
CHUNK = PCHUNK * BATCH
NBUF = 3
CH_PER_PASS = POS_PER_W // PCHUNK
NCHUNK = NPASS * CH_PER_PASS
LANES = 16
NSLICE = D_MODEL // LANES


def _positional_table() -> np.ndarray:
    pos = np.arange(SEQ_LEN, dtype=np.float32)[:, None]
    i = np.arange(D_MODEL, dtype=np.float32)[None, :]
    angle_rates = 1.0 / np.power(10000.0, (2.0 * np.floor(i / 2.0)) / D_MODEL)
    angles = pos * angle_rates
    pe = np.zeros((SEQ_LEN, D_MODEL), dtype=np.float32)
    pe[:, 0::2] = np.sin(angles[:, 0::2])
    pe[:, 1::2] = np.cos(angles[:, 1::2])
    return pe


_PE = _positional_table()


def _sc_body(x_hbm, pe_hbm, table_hbm, out_hbm,
             idx_v, pe_a, pe_b, rows0, rows1, rows2,
             ixs, psa, psb, gs0, gs1, gs2, os0, os1, os2):
    pebuf = (pe_a, pe_b)
    pesem = (psa, psb)
    rows = (rows0, rows1, rows2)
    gsem = (gs0, gs1, gs2)
    osem = (os0, os1, os2)

    c = lax.axis_index("c")
    s = lax.axis_index("s")
    wid = s * 2 + c
    wpos = wid * POS_PER_W

    idx_cps = []
    for p in range(NPASS):
        for b in range(BATCH):
            src_off = b * SEQ_LEN + p * (SEQ_LEN // NPASS) + wpos
            dst_off = (p * BATCH + b) * POS_PER_W
            idx_cps.append(pltpu.async_copy(
                x_hbm.at[pl.ds(src_off, POS_PER_W)],
                idx_v.at[pl.ds(dst_off, POS_PER_W)], ixs))

    def issue_pe(p):
        return pltpu.async_copy(
            pe_hbm.at[pl.ds(p * (SEQ_LEN // NPASS) + wpos, POS_PER_W)],
            pebuf[p % 2], pesem[p % 2])

    pend_pe = {0: issue_pe(0)}

    for cp in idx_cps:
        cp.wait()

    def chunk_coords(ci):
        return ci // CH_PER_PASS, ci % CH_PER_PASS

    def issue_gather(ci, buf):
        p, l = chunk_coords(ci)
        cps = []
        for q in range(BATCH):
            ioff = (p * BATCH + q) * POS_PER_W + l * PCHUNK
            cps.append(pltpu.async_copy(
                table_hbm.at[idx_v.at[pl.ds(ioff, PCHUNK)]],
                rows[buf].at[pl.ds(q * PCHUNK, PCHUNK)], gsem[buf]))
        return cps

    def issue_out(ci, buf):
        p, l = chunk_coords(ci)
        cps = []
        for q in range(BATCH):
            ooff = q * SEQ_LEN + p * (SEQ_LEN // NPASS) + wpos + l * PCHUNK
            cps.append(pltpu.async_copy(
                rows[buf].at[pl.ds(q * PCHUNK, PCHUNK)],
                out_hbm.at[pl.ds(ooff, PCHUNK)], osem[buf]))
        return cps

    pend_g = [issue_gather(0, 0), issue_gather(1, 1)]
    pend_o = [None] * NBUF

    for ci in range(NCHUNK):
        buf = ci % NBUF
        nci = ci + 2
        if nci < NCHUNK:
            nb = nci % NBUF
            if pend_o[nb] is not None:
                for cp in pend_o[nb]:
                    cp.wait()
                pend_o[nb] = None
            pend_g.append(issue_gather(nci, nb))
        for cp in pend_g.pop(0):
            cp.wait()

        p, l = chunk_coords(ci)
        if l == 0:
            if p + 1 < NPASS:
                pend_pe[p + 1] = issue_pe(p + 1)
            pend_pe.pop(p).wait()

        def row_add(i, carry, _pe=pebuf[p % 2], _buf=buf, _l=l):
            def jblock(jj, carry2):
                for u in range(8):
                    sl = pl.ds(jj * (8 * LANES) + u * LANES, LANES)
                    pv = _pe[_l * PCHUNK + i, sl]
                    for q in range(BATCH):
                        plsc.addupdate(rows[_buf].at[q * PCHUNK + i, sl], pv)
                return carry2

            return lax.fori_loop(0, NSLICE // 8, jblock, carry, unroll=False)

        lax.fori_loop(0, PCHUNK, row_add, 0, unroll=False)
        pend_o[buf] = issue_out(ci, buf)

    for bb in range(NBUF):
        if pend_o[bb] is not None:
            for cp in pend_o[bb]:
                cp.wait()


@jax.jit
def _embed(x_flat, pe, table):
    mesh = plsc.VectorSubcoreMesh(core_axis_name="c", subcore_axis_name="s")
    run = pl.kernel(
        _sc_body,
        mesh=mesh,
        out_type=jax.ShapeDtypeStruct((TOK, D_MODEL), jnp.float32),
        scratch_types=[
            pltpu.VMEM((TPW,), jnp.int32),
            pltpu.VMEM((POS_PER_W, D_MODEL), jnp.float32),
            pltpu.VMEM((POS_PER_W, D_MODEL), jnp.float32),
        ] + [pltpu.VMEM((CHUNK, D_MODEL), jnp.float32)] * NBUF
          + [pltpu.SemaphoreType.DMA] * (3 + 2 * NBUF),
    )
    return run(x_flat, pe, table)


def kernel(x, table):
    x_flat = x.reshape(-1).astype(jnp.int32)
    pe = jnp.asarray(_PE)
    out = _embed(x_flat, pe, table)
    return out.reshape(BATCH, SEQ_LEN, D_MODEL)


# speedup vs baseline: 1.0257x; 1.0257x over previous
import numpy as np
import jax
import jax.numpy as jnp
from jax import lax
from jax.experimental import pallas as pl
from jax.experimental.pallas import tpu as pltpu
from jax.experimental.pallas import tpu_sc as plsc

VOCAB = 100000
D_MODEL = 768
MAX_LEN = 8192
BATCH = 4
SEQ_LEN = 4096
PAD_IDX = 1

TOK = BATCH * SEQ_LEN
NUM_WORKERS = 32
TPW = TOK // NUM_WORKERS
NPASS = 4
POS_PER_W = SEQ_LEN // NPASS // NUM_WORKERS
PCHUNK = 8
CHUNK = PCHUNK * BATCH
NBUF = 3
CH_PER_PASS = POS_PER_W // PCHUNK
NCHUNK = NPASS * CH_PER_PASS
LANES = 16
NSLICE = D_MODEL // LANES


def _positional_table() -> np.ndarray:
    pos = np.arange(SEQ_LEN, dtype=np.float32)[:, None]
    i = np.arange(D_MODEL, dtype=np.float32)[None, :]
    angle_rates = 1.0 / np.power(10000.0, (2.0 * np.floor(i / 2.0)) / D_MODEL)
    angles = pos * angle_rates
    pe = np.zeros((SEQ_LEN, D_MODEL), dtype=np.float32)
    pe[:, 0::2] = np.sin(angles[:, 0::2])
    pe[:, 1::2] = np.cos(angles[:, 1::2])
    return pe


_PE = _positional_table()


def _sc_body(x_hbm, pe_hbm, table_hbm, out_hbm,
             idx_v, pe_a, pe_b, rows0, rows1, rows2,
             ixs, psa, psb, gs0, gs1, gs2, os0, os1, os2):
    pebuf = (pe_a, pe_b)
    pesem = (psa, psb)
    rows = (rows0, rows1, rows2)
    gsem = (gs0, gs1, gs2)
    osem = (os0, os1, os2)

    c = lax.axis_index("c")
    s = lax.axis_index("s")
    wid = s * 2 + c
    wpos = wid * POS_PER_W

    idx_cps = []
    for p in range(NPASS):
        for b in range(BATCH):
            src_off = b * SEQ_LEN + p * (SEQ_LEN // NPASS) + wpos
            dst_off = (p * BATCH + b) * POS_PER_W
            idx_cps.append(pltpu.async_copy(
                x_hbm.at[pl.ds(src_off, POS_PER_W)],
                idx_v.at[pl.ds(dst_off, POS_PER_W)], ixs))

    def issue_pe(p):
        return pltpu.async_copy(
            pe_hbm.at[pl.ds(p * (SEQ_LEN // NPASS) + wpos, POS_PER_W)],
            pebuf[p % 2], pesem[p % 2])

    pend_pe = {0: issue_pe(0)}

    for cp in idx_cps:
        cp.wait()

    def chunk_coords(ci):
        return ci // CH_PER_PASS, ci % CH_PER_PASS

    def issue_gather(ci, buf):
        p, l = chunk_coords(ci)
        cps = []
        for q in range(BATCH):
            ioff = (p * BATCH + q) * POS_PER_W + l * PCHUNK
            cps.append(pltpu.async_copy(
                table_hbm.at[idx_v.at[pl.ds(ioff, PCHUNK)]],
                rows[buf].at[pl.ds(q * PCHUNK, PCHUNK)], gsem[buf]))
        return cps

    def issue_out(ci, buf):
        p, l = chunk_coords(ci)
        cps = []
        for q in range(BATCH):
            ooff = q * SEQ_LEN + p * (SEQ_LEN // NPASS) + wpos + l * PCHUNK
            cps.append(pltpu.async_copy(
                rows[buf].at[pl.ds(q * PCHUNK, PCHUNK)],
                out_hbm.at[pl.ds(ooff, PCHUNK)], osem[buf]))
        return cps

    pend_g = [issue_gather(0, 0), issue_gather(1, 1)]
    pend_o = [None] * NBUF

    for ci in range(NCHUNK):
        buf = ci % NBUF
        nci = ci + 2
        if nci < NCHUNK:
            nb = nci % NBUF
            if pend_o[nb] is not None:
                for cp in pend_o[nb]:
                    cp.wait()
                pend_o[nb] = None
            pend_g.append(issue_gather(nci, nb))
        for cp in pend_g.pop(0):
            cp.wait()

        p, l = chunk_coords(ci)
        if l == 0:
            if p + 1 < NPASS:
                pend_pe[p + 1] = issue_pe(p + 1)
            pend_pe.pop(p).wait()

        def row_add(i, carry, _pe=pebuf[p % 2], _buf=buf, _l=l):
            def jblock(jj, carry2):
                for u in range(4):
                    sl = pl.ds(jj * (4 * LANES) + u * LANES, LANES)
                    pv = _pe[_l * PCHUNK + i, sl]
                    for q in range(BATCH):
                        plsc.addupdate(rows[_buf].at[q * PCHUNK + i, sl], pv)
                return carry2

            return lax.fori_loop(0, NSLICE // 4, jblock, carry, unroll=False)

        lax.fori_loop(0, PCHUNK, row_add, 0, unroll=False)
        pend_o[buf] = issue_out(ci, buf)

    for bb in range(NBUF):
        if pend_o[bb] is not None:
            for cp in pend_o[bb]:
                cp.wait()


@jax.jit
def _embed(x_flat, pe, table):
    mesh = plsc.VectorSubcoreMesh(core_axis_name="c", subcore_axis_name="s")
    run = pl.kernel(
        _sc_body,
        mesh=mesh,
        out_type=jax.ShapeDtypeStruct((TOK, D_MODEL), jnp.float32),
        scratch_types=[
            pltpu.VMEM((TPW,), jnp.int32),
            pltpu.VMEM((POS_PER_W, D_MODEL), jnp.float32),
            pltpu.VMEM((POS_PER_W, D_MODEL), jnp.float32),
        ] + [pltpu.VMEM((CHUNK, D_MODEL), jnp.float32)] * NBUF
          + [pltpu.SemaphoreType.DMA] * (3 + 2 * NBUF),
    )
    return run(x_flat, pe, table)


def kernel(x, table):
    x_flat = x.reshape(-1).astype(jnp.int32)
    pe = jnp.asarray(_PE)
    out = _embed(x_flat, pe, table)
    return out.reshape(BATCH, SEQ_LEN, D_MODEL)


# speedup vs baseline: 1.1363x; 1.1078x over previous
import numpy as np
import jax
import jax.numpy as jnp
from jax import lax
from jax.experimental import pallas as pl
from jax.experimental.pallas import tpu as pltpu
from jax.experimental.pallas import tpu_sc as plsc

VOCAB = 100000
D_MODEL = 768
MAX_LEN = 8192
BATCH = 4
SEQ_LEN = 4096
PAD_IDX = 1

TOK = BATCH * SEQ_LEN
NUM_WORKERS = 32
TPW = TOK // NUM_WORKERS
NPASS = 8
POS_PER_W = SEQ_LEN // NPASS // NUM_WORKERS
PCHUNK = 8
CHUNK = PCHUNK * BATCH
NBUF = 4
CH_PER_PASS = POS_PER_W // PCHUNK
NCHUNK = NPASS * CH_PER_PASS
LANES = 16
NSLICE = D_MODEL // LANES


def _positional_table() -> np.ndarray:
    pos = np.arange(SEQ_LEN, dtype=np.float32)[:, None]
    i = np.arange(D_MODEL, dtype=np.float32)[None, :]
    angle_rates = 1.0 / np.power(10000.0, (2.0 * np.floor(i / 2.0)) / D_MODEL)
    angles = pos * angle_rates
    pe = np.zeros((SEQ_LEN, D_MODEL), dtype=np.float32)
    pe[:, 0::2] = np.sin(angles[:, 0::2])
    pe[:, 1::2] = np.cos(angles[:, 1::2])
    return pe


_PE = _positional_table()


def _sc_body(x_hbm, pe_hbm, table_hbm, out_hbm,
             idx_v, pe_a, pe_b, rows0, rows1, rows2, rows3,
             ixs, psa, psb, gs0, gs1, gs2, gs3, os0, os1, os2, os3):
    pebuf = (pe_a, pe_b)
    pesem = (psa, psb)
    rows = (rows0, rows1, rows2, rows3)
    gsem = (gs0, gs1, gs2, gs3)
    osem = (os0, os1, os2, os3)

    c = lax.axis_index("c")
    s = lax.axis_index("s")
    wid = s * 2 + c
    wpos = wid * POS_PER_W

    idx_cps = []
    for p in range(NPASS):
        for b in range(BATCH):
            src_off = b * SEQ_LEN + p * (SEQ_LEN // NPASS) + wpos
            dst_off = (p * BATCH + b) * POS_PER_W
            idx_cps.append(pltpu.async_copy(
                x_hbm.at[pl.ds(src_off, POS_PER_W)],
                idx_v.at[pl.ds(dst_off, POS_PER_W)], ixs))

    def issue_pe(p):
        return pltpu.async_copy(
            pe_hbm.at[pl.ds(p * (SEQ_LEN // NPASS) + wpos, POS_PER_W)],
            pebuf[p % 2], pesem[p % 2])

    pend_pe = {0: issue_pe(0)}

    for cp in idx_cps:
        cp.wait()

    def chunk_coords(ci):
        return ci // CH_PER_PASS, ci % CH_PER_PASS

    def issue_gather(ci, buf):
        p, l = chunk_coords(ci)
        cps = []
        for q in range(BATCH):
            ioff = (p * BATCH + q) * POS_PER_W + l * PCHUNK
            cps.append(pltpu.async_copy(
                table_hbm.at[idx_v.at[pl.ds(ioff, PCHUNK)]],
                rows[buf].at[pl.ds(q * PCHUNK, PCHUNK)], gsem[buf]))
        return cps

    def issue_out(ci, buf):
        p, l = chunk_coords(ci)
        cps = []
        for q in range(BATCH):
            ooff = q * SEQ_LEN + p * (SEQ_LEN // NPASS) + wpos + l * PCHUNK
            cps.append(pltpu.async_copy(
                rows[buf].at[pl.ds(q * PCHUNK, PCHUNK)],
                out_hbm.at[pl.ds(ooff, PCHUNK)], osem[buf]))
        return cps

    pend_g = [issue_gather(0, 0), issue_gather(1, 1)]
    pend_o = [None] * NBUF

    for ci in range(NCHUNK):
        buf = ci % NBUF
        nci = ci + 2
        if nci < NCHUNK:
            nb = nci % NBUF
            if pend_o[nb] is not None:
                for cp in pend_o[nb]:
                    cp.wait()
                pend_o[nb] = None
            pend_g.append(issue_gather(nci, nb))
        for cp in pend_g.pop(0):
            cp.wait()

        p, l = chunk_coords(ci)
        if l == 0:
            if p + 1 < NPASS:
                pend_pe[p + 1] = issue_pe(p + 1)
            pend_pe.pop(p).wait()

        def row_add(i, carry, _pe=pebuf[p % 2], _buf=buf, _l=l):
            def jblock(jj, carry2):
                for u in range(4):
                    sl = pl.ds(jj * (4 * LANES) + u * LANES, LANES)
                    pv = _pe[_l * PCHUNK + i, sl]
                    for q in range(BATCH):
                        plsc.addupdate(rows[_buf].at[q * PCHUNK + i, sl], pv)
                return carry2

            return lax.fori_loop(0, NSLICE // 4, jblock, carry, unroll=False)

        lax.fori_loop(0, PCHUNK, row_add, 0, unroll=False)
        pend_o[buf] = issue_out(ci, buf)

    for bb in range(NBUF):
        if pend_o[bb] is not None:
            for cp in pend_o[bb]:
                cp.wait()


@jax.jit
def _embed(x_flat, pe, table):
    mesh = plsc.VectorSubcoreMesh(core_axis_name="c", subcore_axis_name="s")
    run = pl.kernel(
        _sc_body,
        mesh=mesh,
        out_type=jax.ShapeDtypeStruct((TOK, D_MODEL), jnp.float32),
        scratch_types=[
            pltpu.VMEM((TPW,), jnp.int32),
            pltpu.VMEM((POS_PER_W, D_MODEL), jnp.float32),
            pltpu.VMEM((POS_PER_W, D_MODEL), jnp.float32),
        ] + [pltpu.VMEM((CHUNK, D_MODEL), jnp.float32)] * NBUF
          + [pltpu.SemaphoreType.DMA] * (3 + 2 * NBUF),
    )
    return run(x_flat, pe, table)


def kernel(x, table):
    x_flat = x.reshape(-1).astype(jnp.int32)
    pe = jnp.asarray(_PE)
    out = _embed(x_flat, pe, table)
    return out.reshape(BATCH, SEQ_LEN, D_MODEL)
